# Initial kernel scaffold; baseline (speedup 1.0000x reference)
import jax
import jax.numpy as jnp
from jax.experimental import pallas as pl


def kernel(x):
    raise NotImplementedError("write your pallas kernel here")


# speedup vs baseline: 29.1535x; 29.1535x over previous
import jax
import jax.numpy as jnp
import numpy as np
from jax.experimental import pallas as pl

_SCALE = float(2.0 ** -2.5)


def _body(x_ref, s_ref, o_ref):
    o_ref[...] = jax.lax.dot_general(
        x_ref[...], s_ref[...],
        dimension_numbers=(((1,), (0,)), ((), ())),
        preferred_element_type=jnp.float32,
        precision=jax.lax.Precision.HIGHEST,
    )


def kernel(x):
    B, T = x.shape
    F = T // 32
    R = B * T // 128
    xr = x.reshape(R, 128)

    s = (np.arange(128)[:, None] // 32 == np.arange(4)[None, :])
    s = jnp.asarray(s.astype(np.float32) * _SCALE)

    BLK = 2048
    out = pl.pallas_call(
        _body,
        grid=(R // BLK,),
        in_specs=[
            pl.BlockSpec((BLK, 128), lambda i: (i, 0)),
            pl.BlockSpec((128, 4), lambda i: (0, 0)),
        ],
        out_specs=pl.BlockSpec((BLK, 4), lambda i: (i, 0)),
        out_shape=jax.ShapeDtypeStruct((R, 4), jnp.float32),
    )(xr, s)
    return out.reshape(B, F, 1)


# speedup vs baseline: 31.9577x; 1.0962x over previous
import jax
import jax.numpy as jnp
import numpy as np
from jax.experimental import pallas as pl

_SCALE = float(2.0 ** -2.5)


def _body(x_ref, s_ref, o_ref):
    o_ref[...] = jax.lax.dot_general(
        x_ref[...], s_ref[...],
        dimension_numbers=(((1,), (0,)), ((), ())),
        preferred_element_type=jnp.float32,
        precision=jax.lax.Precision.HIGHEST,
    )


def kernel(x):
    B, T = x.shape
    F = T // 32
    R = B * T // 128
    xr = x.reshape(R, 128)

    s = (np.arange(128)[:, None] // 32 == np.arange(4)[None, :])
    s = jnp.asarray(s.astype(np.float32) * _SCALE)

    BLK = 8192
    out = pl.pallas_call(
        _body,
        grid=(R // BLK,),
        in_specs=[
            pl.BlockSpec((BLK, 128), lambda i: (i, 0)),
            pl.BlockSpec((128, 4), lambda i: (0, 0)),
        ],
        out_specs=pl.BlockSpec((BLK, 4), lambda i: (i, 0)),
        out_shape=jax.ShapeDtypeStruct((R, 4), jnp.float32),
    )(xr, s)
    return out.reshape(B, F, 1)


# speedup vs baseline: 49.1550x; 1.5381x over previous
import jax
import jax.numpy as jnp
import numpy as np
from jax.experimental import pallas as pl

_SCALE = float(2.0 ** -2.5)


def _body(x_ref, o_ref):
    xb = x_ref[...]
    r = xb.shape[0]
    o_ref[...] = xb.reshape(r, xb.shape[1] // 32, 32).sum(axis=-1) * _SCALE


def kernel(x):
    B, T = x.shape
    F = T // 32
    RB = 16
    out = pl.pallas_call(
        _body,
        grid=(B // RB,),
        in_specs=[pl.BlockSpec((RB, T), lambda i: (i, 0))],
        out_specs=pl.BlockSpec((RB, F), lambda i: (i, 0)),
        out_shape=jax.ShapeDtypeStruct((B, F), jnp.float32),
    )(x)
    return out[:, :, None]
